# baseline (speedup 1.0000x reference)
import functools
import math
import jax, jax.numpy as jnp
from jax import lax
from jax.experimental import pallas as pl
from jax.experimental.pallas import tpu as pltpu
from jax.experimental.pallas import tpu_sc as plsc

RATIO = 0.5

N0 = 10000
E0 = 320000
K1 = 5000
NR = 5120
NC0 = 10240
SENT = NR - 1
EEXT = 327680
SLAB = 1638400
NSLAB = 32
TPS = SLAB // 16
NCHUNK = 160
EAGG = 327680
NAGG = 10112
AHALF = 5056
AROWS = 5120

_mesh = plsc.VectorSubcoreMesh(core_axis_name="c", subcore_axis_name="s")


def _bc_body(fbc, zslab, ones_h, out, slab, idx2d, ones_v, sem):
    core = lax.axis_index("c")
    tid = lax.axis_index("s")
    pltpu.sync_copy(ones_h, ones_v)

    def slab_body(s, carry):
        base = s * SLAB
        pltpu.sync_copy(zslab, slab.at[pl.ds(tid * TPS, TPS)])
        pltpu.sync_copy(fbc.at[core, tid], idx2d)
        plsc.subcore_barrier()

        def chunk_body(j, c2):
            for q in range(8):
                v = idx2d[j, pl.ds(q * 16, 16)]
                t = v - base
                ok = (t >= 0) & (t < SLAB)
                trash = SLAB + lax.iota(jnp.int32, 16) * 8 + q
                idx2d[j, pl.ds(q * 16, 16)] = jnp.where(ok, t, trash)
            pltpu.async_copy(ones_v, slab.at[idx2d.at[j]], sem, add=True)

            @pl.when(j >= 16)
            def _():
                pltpu.make_async_copy(ones_v, slab.at[idx2d.at[0]], sem).wait()
            return c2

        lax.fori_loop(0, NCHUNK, chunk_body, 0, unroll=False)
        for _ in range(16):
            pltpu.make_async_copy(ones_v, slab.at[idx2d.at[0]], sem).wait()
        plsc.subcore_barrier()
        pltpu.sync_copy(slab.at[pl.ds(tid * TPS, TPS)],
                        out.at[core, pl.ds(base + tid * TPS, TPS)])
        plsc.subcore_barrier()
        return carry

    lax.fori_loop(0, NSLAB, slab_body, 0, unroll=False)


def _make_bc_builder():
    return pl.kernel(
        _bc_body,
        out_type=jax.ShapeDtypeStruct((2, NR * NC0), jnp.float32),
        mesh=_mesh,
        scratch_types=[
            pltpu.VMEM_SHARED((SLAB + 128,), jnp.float32),
            pltpu.VMEM((NCHUNK, 128), jnp.int32),
            pltpu.VMEM((128,), jnp.float32),
            pltpu.SemaphoreType.DMA,
        ],
    )


def _agg_body(width, z, sidx, didx, zrows, out, accum, si, di, rows, gsem):
    core = lax.axis_index("c")
    tid = lax.axis_index("s")
    base = core * AHALF
    pltpu.sync_copy(sidx.at[tid], si)
    pltpu.sync_copy(didx.at[tid], di)
    nrt = AROWS // 16
    pltpu.sync_copy(zrows, accum.at[pl.ds(tid * nrt, nrt)])
    plsc.subcore_barrier()

    def chunk_body(j, c):
        for q in range(8):
            v = di[j, pl.ds(q * 16, 16)]
            t = v - base
            ok = (t >= 0) & (t < AHALF)
            trash = AHALF + lax.iota(jnp.int32, 16) * 4 + (q & 3)
            di[j, pl.ds(q * 16, 16)] = jnp.where(ok, t, trash)
        pltpu.async_copy(z.at[si.at[j]], rows, gsem).wait()
        pltpu.sync_copy(rows, accum.at[di.at[j]], add=True)
        return c

    lax.fori_loop(0, 160, chunk_body, 0, unroll=False)
    plsc.subcore_barrier()
    pltpu.sync_copy(accum.at[pl.ds(tid * nrt, nrt)],
                    out.at[core, pl.ds(tid * nrt, nrt)])


def _make_agg(width):
    return pl.kernel(
        functools.partial(_agg_body, width),
        out_type=jax.ShapeDtypeStruct((2, AROWS, width), jnp.float32),
        mesh=_mesh,
        scratch_types=[
            pltpu.VMEM_SHARED((AROWS, width), jnp.float32),
            pltpu.VMEM((160, 128), jnp.int32),
            pltpu.VMEM((160, 128), jnp.int32),
            pltpu.VMEM((128, width), jnp.float32),
            pltpu.SemaphoreType.DMA,
        ],
    )


def _sq_mm(G1, G2, bm, bn, bk, cast_bf16=True):
    M, K = G1.shape
    _, N = G2.shape
    gm, gn, gk = M // bm, N // bn, K // bk

    def body(g1_ref, g2_ref, of_ref, ob_ref):
        k = pl.program_id(2)

        @pl.when(k == 0)
        def _():
            of_ref[...] = jnp.zeros_like(of_ref)

        if cast_bf16:
            of_ref[...] += jnp.dot(g1_ref[...].astype(jnp.bfloat16),
                                   g2_ref[...].astype(jnp.bfloat16),
                                   preferred_element_type=jnp.float32)
        else:
            of_ref[...] += jnp.dot(g1_ref[...], g2_ref[...],
                                   preferred_element_type=jnp.float32)

        @pl.when(k == gk - 1)
        def _():
            m = pl.program_id(0)
            n = pl.program_id(1)
            rows = lax.broadcasted_iota(jnp.int32, (bm, bn), 0) + m * bm
            cols = lax.broadcasted_iota(jnp.int32, (bm, bn), 1) + n * bn
            sent = (rows == M - 1) | (cols == N - 1)
            res = jnp.where(rows == cols, 1.0,
                            jnp.where(sent, 0.0, of_ref[...]))
            of_ref[...] = res
            ob_ref[...] = res.astype(jnp.bfloat16)

    return pl.pallas_call(
        body,
        grid=(gm, gn, gk),
        in_specs=[
            pl.BlockSpec((bm, bk), lambda m, n, k: (m, k)),
            pl.BlockSpec((bk, bn), lambda m, n, k: (k, n)),
        ],
        out_specs=[
            pl.BlockSpec((bm, bn), lambda m, n, k: (m, n)),
            pl.BlockSpec((bm, bn), lambda m, n, k: (m, n)),
        ],
        out_shape=[
            jax.ShapeDtypeStruct((M, N), jnp.float32),
            jax.ShapeDtypeStruct((M, N), jnp.bfloat16),
        ],
        compiler_params=pltpu.CompilerParams(
            dimension_semantics=("parallel", "parallel", "arbitrary")),
    )(G1, G2)


def kernel(x, edge_index, W_down0, b_down0, W_down1, b_down1, W_down2, b_down2,
           W_down3, b_down3, w_pool0, w_pool1, w_pool2,
           W_up0, b_up0, W_up1, b_up1, W_up2, b_up2):
    n0 = x.shape[0]
    src, dst = edge_index[0], edge_index[1]
    f32 = jnp.float32

    deg0 = 2.0 + jnp.zeros((n0,), f32).at[dst].add(1.0)
    dinv0 = jnp.where(deg0 > 0.0, deg0 ** -0.5, 0.0)

    npad = EAGG - E0
    srcp = jnp.concatenate([src, jnp.full((npad,), N0, jnp.int32)])
    dstp = jnp.concatenate(
        [dst, N0 + (jnp.arange(npad, dtype=jnp.int32) & 15)])
    sidx = srcp.reshape(16, 160, 128)
    didx = dstp.reshape(16, 160, 128)
    zrows = jnp.zeros((AROWS // 16, 128), f32)
    agg128 = _make_agg(128)

    def gcn0(xin, W, b):
        z = dinv0[:, None] * (xin @ W)
        w = W.shape[1]
        zp = jnp.concatenate([z, jnp.zeros((NAGG - n0, w), f32)])
        if w < 128:
            zp = jnp.concatenate([zp, jnp.zeros((NAGG, 128 - w), f32)], axis=1)
        parts = agg128(zp, sidx, didx, zrows)
        agg = jnp.concatenate([parts[0, :AHALF], parts[1, :n0 - AHALF]])[:, :w]
        return dinv0[:, None] * (agg + 2.0 * z) + b

    def pool_score(xin, w):
        return jnp.tanh((xin @ w) / jnp.sqrt(jnp.sum(w * w)))

    x0 = jax.nn.relu(gcn0(x, W_down0, b_down0))

    k1 = int(math.ceil(RATIO * n0))
    s0 = pool_score(x0, w_pool0)
    sv0, perm0 = lax.top_k(s0, k1)
    slot = jnp.full((N0 + 16,), SENT, jnp.int32).at[perm0].set(
        jnp.arange(k1, dtype=jnp.int32))
    epad = EEXT - E0 - k1
    srcx = jnp.concatenate([src, perm0, jnp.full((epad,), N0, jnp.int32)])
    dstx = jnp.concatenate([dst, perm0, jnp.full((epad,), N0, jnp.int32)])
    rb = slot[srcx]
    cb = slot[dstx]
    fb = (rb * NC0 + dstx).reshape(16, NCHUNK, 128)
    fc = (srcx * NR + cb).reshape(16, NCHUNK, 128)
    fbc = jnp.stack([fb, fc])
    zslab = jnp.zeros((TPS,), jnp.float32)
    ones_h = jnp.ones((128,), jnp.float32)
    bc = _make_bc_builder()(fbc, zslab, ones_h)
    B = bc[0].reshape(NR, NC0)
    C = bc[1].reshape(NC0, NR)

    M1f, M1b = _sq_mm(B, C, 512, 512, 1024)
    deg1 = jnp.sum(M1f, axis=0)[:k1] + 1.0
    dinv1 = jnp.where(deg1 > 0.0, deg1 ** -0.5, 0.0)

    def gcn_dense(xin, Mf, dinv, W, b, kv, kpad):
        z = dinv[:, None] * (xin @ W)
        zp = jnp.concatenate([z, jnp.zeros((kpad - kv, W.shape[1]), f32)])
        agg = jnp.einsum('ij,if->jf', Mf, zp,
                         preferred_element_type=f32)[:kv] + z
        return dinv[:, None] * agg + b

    xp0 = x0[perm0] * sv0[:, None]
    x1 = jax.nn.relu(gcn_dense(xp0, M1f, dinv1, W_down1, b_down1, k1, NR))

    k2 = int(math.ceil(RATIO * k1))
    k2p = 2560
    s1 = pool_score(x1, w_pool1)
    sv1, perm1 = lax.top_k(s1, k2)
    perm1p = jnp.concatenate(
        [perm1, jnp.full((k2p - k2,), NR - 1, jnp.int32)])
    M2f, M2b = _sq_mm(M1b[perm1p], M1b[:, perm1p], 512, 512, 1024)
    deg2 = jnp.sum(M2f, axis=0)[:k2] + 1.0
    dinv2 = jnp.where(deg2 > 0.0, deg2 ** -0.5, 0.0)
    xp1 = x1[perm1] * sv1[:, None]
    x2 = jax.nn.relu(gcn_dense(xp1, M2f, dinv2, W_down2, b_down2, k2, k2p))

    k3 = int(math.ceil(RATIO * k2))
    k3p = 1280
    s2 = pool_score(x2, w_pool2)
    sv2, perm2 = lax.top_k(s2, k3)
    perm2p = jnp.concatenate(
        [perm2, jnp.full((k3p - k3,), k2p - 1, jnp.int32)])
    M3f, _ = _sq_mm(M2f[perm2p], M2f[:, perm2p], 256, 256, 512,
                    cast_bf16=False)
    deg3 = jnp.sum(M3f, axis=0)[:k3] + 1.0
    dinv3 = jnp.where(deg3 > 0.0, deg3 ** -0.5, 0.0)
    xp2 = x2[perm2] * sv2[:, None]
    x3 = jax.nn.relu(gcn_dense(xp2, M3f, dinv3, W_down3, b_down3, k3, k3p))

    u2 = x2 + jnp.zeros_like(x2).at[perm2].set(x3)
    xu = jax.nn.relu(gcn_dense(u2, M2f, dinv2, W_up0, b_up0, k2, k2p))
    u1 = x1 + jnp.zeros_like(x1).at[perm1].set(xu)
    xu = jax.nn.relu(gcn_dense(u1, M1f, dinv1, W_up1, b_up1, k1, NR))
    u0 = x0 + jnp.zeros((n0, xu.shape[1]), f32).at[perm0].set(xu)
    return gcn0(u0, W_up2, b_up2)


# speedup vs baseline: 1.2713x; 1.2713x over previous
import functools
import math
import jax, jax.numpy as jnp
from jax import lax
from jax.experimental import pallas as pl
from jax.experimental.pallas import tpu as pltpu
from jax.experimental.pallas import tpu_sc as plsc

RATIO = 0.5

N0 = 10000
E0 = 320000
K1 = 5000
NR = 5120
NC0 = 10240
SENT = NR - 1
EEXT = 327680
SLAB = 1638400
NSLAB = 32
TPS = SLAB // 16
NCHUNK = 160
EAGG = 327680
NAGG = 10112
AHALF = 5056
AROWS = 5120

_mesh = plsc.VectorSubcoreMesh(core_axis_name="c", subcore_axis_name="s")


def _bc_body(fbc, zslab, ones_h, out, slab, idx2d, ones_v, sem):
    core = lax.axis_index("c")
    tid = lax.axis_index("s")
    pltpu.sync_copy(ones_h, ones_v)

    def slab_body(s, carry):
        base = s * SLAB
        pltpu.sync_copy(zslab, slab.at[pl.ds(tid * TPS, TPS)])
        pltpu.sync_copy(fbc.at[core, tid], idx2d)
        plsc.subcore_barrier()

        def chunk_body(j, c2):
            for q in range(8):
                v = idx2d[j, pl.ds(q * 16, 16)]
                t = v - base
                ok = (t >= 0) & (t < SLAB)
                trash = SLAB + lax.iota(jnp.int32, 16) * 8 + q
                idx2d[j, pl.ds(q * 16, 16)] = jnp.where(ok, t, trash)
            pltpu.async_copy(ones_v, slab.at[idx2d.at[j]], sem, add=True)

            @pl.when(j >= 16)
            def _():
                pltpu.make_async_copy(ones_v, slab.at[idx2d.at[0]], sem).wait()
            return c2

        lax.fori_loop(0, NCHUNK, chunk_body, 0, unroll=False)
        for _ in range(16):
            pltpu.make_async_copy(ones_v, slab.at[idx2d.at[0]], sem).wait()
        plsc.subcore_barrier()
        pltpu.sync_copy(slab.at[pl.ds(tid * TPS, TPS)],
                        out.at[core, pl.ds(base + tid * TPS, TPS)])
        plsc.subcore_barrier()
        return carry

    lax.fori_loop(0, NSLAB, slab_body, 0, unroll=False)


def _make_bc_builder():
    return pl.kernel(
        _bc_body,
        out_type=jax.ShapeDtypeStruct((2, NR * NC0), jnp.float32),
        mesh=_mesh,
        scratch_types=[
            pltpu.VMEM_SHARED((SLAB + 128,), jnp.float32),
            pltpu.VMEM((NCHUNK, 128), jnp.int32),
            pltpu.VMEM((128,), jnp.float32),
            pltpu.SemaphoreType.DMA,
        ],
    )


def _agg_body(width, z, sidx, didx, zrows, out, accum, si, di, rows, gsem):
    core = lax.axis_index("c")
    tid = lax.axis_index("s")
    base = core * AHALF
    pltpu.sync_copy(sidx.at[tid], si)
    pltpu.sync_copy(didx.at[tid], di)
    nrt = AROWS // 16
    pltpu.sync_copy(zrows, accum.at[pl.ds(tid * nrt, nrt)])
    plsc.subcore_barrier()

    def chunk_body(j, c):
        for q in range(8):
            v = di[j, pl.ds(q * 16, 16)]
            t = v - base
            ok = (t >= 0) & (t < AHALF)
            trash = AHALF + lax.iota(jnp.int32, 16) * 4 + (q & 3)
            di[j, pl.ds(q * 16, 16)] = jnp.where(ok, t, trash)
        pltpu.async_copy(z.at[si.at[j]], rows, gsem).wait()
        pltpu.sync_copy(rows, accum.at[di.at[j]], add=True)
        return c

    lax.fori_loop(0, 160, chunk_body, 0, unroll=False)
    plsc.subcore_barrier()
    pltpu.sync_copy(accum.at[pl.ds(tid * nrt, nrt)],
                    out.at[core, pl.ds(tid * nrt, nrt)])


def _make_agg(width):
    return pl.kernel(
        functools.partial(_agg_body, width),
        out_type=jax.ShapeDtypeStruct((2, AROWS, width), jnp.float32),
        mesh=_mesh,
        scratch_types=[
            pltpu.VMEM_SHARED((AROWS, width), jnp.float32),
            pltpu.VMEM((160, 128), jnp.int32),
            pltpu.VMEM((160, 128), jnp.int32),
            pltpu.VMEM((128, width), jnp.float32),
            pltpu.SemaphoreType.DMA,
        ],
    )


def _sq_mm(G1, G2, bm, bn, bk, cast_bf16=True):
    M, K = G1.shape
    _, N = G2.shape
    gm, gn, gk = M // bm, N // bn, K // bk

    def body(g1_ref, g2_ref, of_ref, ob_ref):
        k = pl.program_id(2)

        @pl.when(k == 0)
        def _():
            of_ref[...] = jnp.zeros_like(of_ref)

        if cast_bf16:
            of_ref[...] += jnp.dot(g1_ref[...].astype(jnp.bfloat16),
                                   g2_ref[...].astype(jnp.bfloat16),
                                   preferred_element_type=jnp.float32)
        else:
            of_ref[...] += jnp.dot(g1_ref[...], g2_ref[...],
                                   preferred_element_type=jnp.float32)

        @pl.when(k == gk - 1)
        def _():
            m = pl.program_id(0)
            n = pl.program_id(1)
            rows = lax.broadcasted_iota(jnp.int32, (bm, bn), 0) + m * bm
            cols = lax.broadcasted_iota(jnp.int32, (bm, bn), 1) + n * bn
            sent = (rows == M - 1) | (cols == N - 1)
            res = jnp.where(rows == cols, 1.0,
                            jnp.where(sent, 0.0, of_ref[...]))
            of_ref[...] = res
            ob_ref[...] = res.astype(jnp.bfloat16)

    return pl.pallas_call(
        body,
        grid=(gm, gn, gk),
        in_specs=[
            pl.BlockSpec((bm, bk), lambda m, n, k: (m, k)),
            pl.BlockSpec((bk, bn), lambda m, n, k: (k, n)),
        ],
        out_specs=[
            pl.BlockSpec((bm, bn), lambda m, n, k: (m, n)),
            pl.BlockSpec((bm, bn), lambda m, n, k: (m, n)),
        ],
        out_shape=[
            jax.ShapeDtypeStruct((M, N), jnp.float32),
            jax.ShapeDtypeStruct((M, N), jnp.bfloat16),
        ],
        compiler_params=pltpu.CompilerParams(
            dimension_semantics=("parallel", "parallel", "arbitrary")),
    )(G1, G2)


def kernel(x, edge_index, W_down0, b_down0, W_down1, b_down1, W_down2, b_down2,
           W_down3, b_down3, w_pool0, w_pool1, w_pool2,
           W_up0, b_up0, W_up1, b_up1, W_up2, b_up2):
    n0 = x.shape[0]
    src, dst = edge_index[0], edge_index[1]
    f32 = jnp.float32
    k1 = 5000
    s0 = jnp.tanh((x @ w_pool0) / jnp.sqrt(jnp.sum(w_pool0 * w_pool0)))
    sv0, perm0 = lax.top_k(s0, k1)
    slot = jnp.full((N0 + 16,), SENT, jnp.int32).at[perm0].set(
        jnp.arange(k1, dtype=jnp.int32))
    epad = EEXT - E0 - k1
    srcx = jnp.concatenate([src, perm0, jnp.full((epad,), N0, jnp.int32)])
    dstx = jnp.concatenate([dst, perm0, jnp.full((epad,), N0, jnp.int32)])
    rb = slot[srcx]
    cb = slot[dstx]
    fb = (rb * NC0 + dstx).reshape(16, NCHUNK, 128)
    fc = (srcx * NR + cb).reshape(16, NCHUNK, 128)
    fbc = jnp.stack([fb, fc])
    zslab = jnp.zeros((TPS,), jnp.float32)
    ones_h = jnp.ones((128,), jnp.float32)
    bc = _make_bc_builder()(fbc, zslab, ones_h)
    B = bc[0].reshape(NR, NC0)
    C = bc[1].reshape(NC0, NR)
    M1f, M1b = _sq_mm(B, C, 512, 512, 1024)
    return jnp.sum(M1f, axis=0)[:k1]


# speedup vs baseline: 1.4502x; 1.1407x over previous
import functools
import math
import jax, jax.numpy as jnp
from jax import lax
from jax.experimental import pallas as pl
from jax.experimental.pallas import tpu as pltpu
from jax.experimental.pallas import tpu_sc as plsc

RATIO = 0.5

N0 = 10000
E0 = 320000
K1 = 5000
NR = 5120
NC0 = 10240
SENT = NR - 1
EEXT = 327680
SLAB = 1638400
NSLAB = 32
TPS = SLAB // 16
NCHUNK = 160
EAGG = 327680
NAGG = 10112
AHALF = 5056
AROWS = 5120

_mesh = plsc.VectorSubcoreMesh(core_axis_name="c", subcore_axis_name="s")


def _bc_body(fbc, zslab, ones_h, out, slab, idx2d, ones_v, sem):
    core = lax.axis_index("c")
    tid = lax.axis_index("s")
    pltpu.sync_copy(ones_h, ones_v)

    def slab_body(s, carry):
        base = s * SLAB
        pltpu.sync_copy(zslab, slab.at[pl.ds(tid * TPS, TPS)])
        pltpu.sync_copy(fbc.at[core, tid], idx2d)
        plsc.subcore_barrier()

        def chunk_body(j, c2):
            for q in range(8):
                v = idx2d[j, pl.ds(q * 16, 16)]
                t = v - base
                ok = (t >= 0) & (t < SLAB)
                trash = SLAB + lax.iota(jnp.int32, 16) * 8 + q
                idx2d[j, pl.ds(q * 16, 16)] = jnp.where(ok, t, trash)
            pltpu.async_copy(ones_v, slab.at[idx2d.at[j]], sem, add=True)

            @pl.when(j >= 16)
            def _():
                pltpu.make_async_copy(ones_v, slab.at[idx2d.at[0]], sem).wait()
            return c2

        lax.fori_loop(0, NCHUNK, chunk_body, 0, unroll=False)
        for _ in range(16):
            pltpu.make_async_copy(ones_v, slab.at[idx2d.at[0]], sem).wait()
        plsc.subcore_barrier()
        pltpu.sync_copy(slab.at[pl.ds(tid * TPS, TPS)],
                        out.at[core, pl.ds(base + tid * TPS, TPS)])
        plsc.subcore_barrier()
        return carry

    lax.fori_loop(0, NSLAB, slab_body, 0, unroll=False)


def _make_bc_builder():
    return pl.kernel(
        _bc_body,
        out_type=jax.ShapeDtypeStruct((2, NR * NC0), jnp.float32),
        mesh=_mesh,
        scratch_types=[
            pltpu.VMEM_SHARED((SLAB + 128,), jnp.float32),
            pltpu.VMEM((NCHUNK, 128), jnp.int32),
            pltpu.VMEM((128,), jnp.float32),
            pltpu.SemaphoreType.DMA,
        ],
    )


def _agg_body(width, z, sidx, didx, zrows, out, accum, si, di, rows, gsem):
    core = lax.axis_index("c")
    tid = lax.axis_index("s")
    base = core * AHALF
    pltpu.sync_copy(sidx.at[tid], si)
    pltpu.sync_copy(didx.at[tid], di)
    nrt = AROWS // 16
    pltpu.sync_copy(zrows, accum.at[pl.ds(tid * nrt, nrt)])
    plsc.subcore_barrier()

    def chunk_body(j, c):
        for q in range(8):
            v = di[j, pl.ds(q * 16, 16)]
            t = v - base
            ok = (t >= 0) & (t < AHALF)
            trash = AHALF + lax.iota(jnp.int32, 16) * 4 + (q & 3)
            di[j, pl.ds(q * 16, 16)] = jnp.where(ok, t, trash)
        pltpu.async_copy(z.at[si.at[j]], rows, gsem).wait()
        pltpu.sync_copy(rows, accum.at[di.at[j]], add=True)
        return c

    lax.fori_loop(0, 160, chunk_body, 0, unroll=False)
    plsc.subcore_barrier()
    pltpu.sync_copy(accum.at[pl.ds(tid * nrt, nrt)],
                    out.at[core, pl.ds(tid * nrt, nrt)])


def _make_agg(width):
    return pl.kernel(
        functools.partial(_agg_body, width),
        out_type=jax.ShapeDtypeStruct((2, AROWS, width), jnp.float32),
        mesh=_mesh,
        scratch_types=[
            pltpu.VMEM_SHARED((AROWS, width), jnp.float32),
            pltpu.VMEM((160, 128), jnp.int32),
            pltpu.VMEM((160, 128), jnp.int32),
            pltpu.VMEM((128, width), jnp.float32),
            pltpu.SemaphoreType.DMA,
        ],
    )


def _sq_mm(G1, G2, bm, bn, bk, cast_bf16=True):
    M, K = G1.shape
    _, N = G2.shape
    gm, gn, gk = M // bm, N // bn, K // bk

    def body(g1_ref, g2_ref, of_ref, ob_ref):
        k = pl.program_id(2)

        @pl.when(k == 0)
        def _():
            of_ref[...] = jnp.zeros_like(of_ref)

        if cast_bf16:
            of_ref[...] += jnp.dot(g1_ref[...].astype(jnp.bfloat16),
                                   g2_ref[...].astype(jnp.bfloat16),
                                   preferred_element_type=jnp.float32)
        else:
            of_ref[...] += jnp.dot(g1_ref[...], g2_ref[...],
                                   preferred_element_type=jnp.float32)

        @pl.when(k == gk - 1)
        def _():
            m = pl.program_id(0)
            n = pl.program_id(1)
            rows = lax.broadcasted_iota(jnp.int32, (bm, bn), 0) + m * bm
            cols = lax.broadcasted_iota(jnp.int32, (bm, bn), 1) + n * bn
            sent = (rows == M - 1) | (cols == N - 1)
            res = jnp.where(rows == cols, 1.0,
                            jnp.where(sent, 0.0, of_ref[...]))
            of_ref[...] = res
            ob_ref[...] = res.astype(jnp.bfloat16)

    return pl.pallas_call(
        body,
        grid=(gm, gn, gk),
        in_specs=[
            pl.BlockSpec((bm, bk), lambda m, n, k: (m, k)),
            pl.BlockSpec((bk, bn), lambda m, n, k: (k, n)),
        ],
        out_specs=[
            pl.BlockSpec((bm, bn), lambda m, n, k: (m, n)),
            pl.BlockSpec((bm, bn), lambda m, n, k: (m, n)),
        ],
        out_shape=[
            jax.ShapeDtypeStruct((M, N), jnp.float32),
            jax.ShapeDtypeStruct((M, N), jnp.bfloat16),
        ],
        compiler_params=pltpu.CompilerParams(
            dimension_semantics=("parallel", "parallel", "arbitrary")),
    )(G1, G2)


def kernel(x, edge_index, W_down0, b_down0, W_down1, b_down1, W_down2, b_down2,
           W_down3, b_down3, w_pool0, w_pool1, w_pool2,
           W_up0, b_up0, W_up1, b_up1, W_up2, b_up2):
    n0 = x.shape[0]
    src, dst = edge_index[0], edge_index[1]
    f32 = jnp.float32

    deg0 = 2.0 + jnp.zeros((n0,), f32).at[dst].add(1.0)
    dinv0 = jnp.where(deg0 > 0.0, deg0 ** -0.5, 0.0)

    npad = EAGG - E0
    srcp = jnp.concatenate([src, jnp.full((npad,), N0, jnp.int32)])
    dstp = jnp.concatenate(
        [dst, N0 + (jnp.arange(npad, dtype=jnp.int32) & 15)])
    sidx = srcp.reshape(16, 160, 128)
    didx = dstp.reshape(16, 160, 128)
    zrows = jnp.zeros((AROWS // 16, 128), f32)
    agg128 = _make_agg(128)

    def gcn0(xin, W, b):
        z = dinv0[:, None] * (xin @ W)
        w = W.shape[1]
        zp = jnp.concatenate([z, jnp.zeros((NAGG - n0, w), f32)])
        if w < 128:
            zp = jnp.concatenate([zp, jnp.zeros((NAGG, 128 - w), f32)], axis=1)
        parts = agg128(zp, sidx, didx, zrows)
        agg = jnp.concatenate([parts[0, :AHALF], parts[1, :n0 - AHALF]])[:, :w]
        return dinv0[:, None] * (agg + 2.0 * z) + b

    def pool_score(xin, w):
        return jnp.tanh((xin @ w) / jnp.sqrt(jnp.sum(w * w)))

    x0 = jax.nn.relu(gcn0(x, W_down0, b_down0))

    k1 = int(math.ceil(RATIO * n0))
    s0 = pool_score(x0, w_pool0)
    sv0, perm0 = lax.top_k(s0, k1)
    slot = jnp.full((N0 + 16,), SENT, jnp.int32).at[perm0].set(
        jnp.arange(k1, dtype=jnp.int32))
    epad = EEXT - E0 - k1
    srcx = jnp.concatenate([src, perm0, jnp.full((epad,), N0, jnp.int32)])
    dstx = jnp.concatenate([dst, perm0, jnp.full((epad,), N0, jnp.int32)])
    slotmat = jnp.broadcast_to(slot[:, None], (N0 + 16, 128))
    rb = slotmat[srcx][:, 0]
    cb = slotmat[dstx][:, 0]
    fb = (rb * NC0 + dstx).reshape(16, NCHUNK, 128)
    fc = (srcx * NR + cb).reshape(16, NCHUNK, 128)
    fbc = jnp.stack([fb, fc])
    zslab = jnp.zeros((TPS,), jnp.float32)
    ones_h = jnp.ones((128,), jnp.float32)
    bc = _make_bc_builder()(fbc, zslab, ones_h)
    B = bc[0].reshape(NR, NC0)
    C = bc[1].reshape(NC0, NR)

    M1f, M1b = _sq_mm(B, C, 512, 512, 1024)
    deg1 = jnp.sum(M1f, axis=0)[:k1] + 1.0
    dinv1 = jnp.where(deg1 > 0.0, deg1 ** -0.5, 0.0)

    def gcn_dense(xin, Mf, dinv, W, b, kv, kpad):
        z = dinv[:, None] * (xin @ W)
        zp = jnp.concatenate([z, jnp.zeros((kpad - kv, W.shape[1]), f32)])
        agg = jnp.einsum('ij,if->jf', Mf, zp,
                         preferred_element_type=f32)[:kv] + z
        return dinv[:, None] * agg + b

    xp0 = x0[perm0] * sv0[:, None]
    x1 = jax.nn.relu(gcn_dense(xp0, M1f, dinv1, W_down1, b_down1, k1, NR))

    k2 = int(math.ceil(RATIO * k1))
    k2p = 2560
    s1 = pool_score(x1, w_pool1)
    sv1, perm1 = lax.top_k(s1, k2)
    perm1p = jnp.concatenate(
        [perm1, jnp.full((k2p - k2,), NR - 1, jnp.int32)])
    M2f, M2b = _sq_mm(M1b[perm1p], M1b[:, perm1p], 512, 512, 1024)
    deg2 = jnp.sum(M2f, axis=0)[:k2] + 1.0
    dinv2 = jnp.where(deg2 > 0.0, deg2 ** -0.5, 0.0)
    xp1 = x1[perm1] * sv1[:, None]
    x2 = jax.nn.relu(gcn_dense(xp1, M2f, dinv2, W_down2, b_down2, k2, k2p))

    k3 = int(math.ceil(RATIO * k2))
    k3p = 1280
    s2 = pool_score(x2, w_pool2)
    sv2, perm2 = lax.top_k(s2, k3)
    perm2p = jnp.concatenate(
        [perm2, jnp.full((k3p - k3,), k2p - 1, jnp.int32)])
    M3f, _ = _sq_mm(M2f[perm2p], M2f[:, perm2p], 256, 256, 512,
                    cast_bf16=False)
    deg3 = jnp.sum(M3f, axis=0)[:k3] + 1.0
    dinv3 = jnp.where(deg3 > 0.0, deg3 ** -0.5, 0.0)
    xp2 = x2[perm2] * sv2[:, None]
    x3 = jax.nn.relu(gcn_dense(xp2, M3f, dinv3, W_down3, b_down3, k3, k3p))

    u2 = x2 + jnp.zeros_like(x2).at[perm2].set(x3)
    xu = jax.nn.relu(gcn_dense(u2, M2f, dinv2, W_up0, b_up0, k2, k2p))
    u1 = x1 + jnp.zeros_like(x1).at[perm1].set(xu)
    xu = jax.nn.relu(gcn_dense(u1, M1f, dinv1, W_up1, b_up1, k1, NR))
    u0 = x0 + jnp.zeros((n0, xu.shape[1]), f32).at[perm0].set(xu)
    return gcn0(u0, W_up2, b_up2)
